# Initial kernel scaffold; baseline (speedup 1.0000x reference)
import functools

import jax
import jax.numpy as jnp
from jax import lax
from jax.experimental import pallas as pl
from jax.experimental.pallas import tpu as pltpu
from jax.experimental.pallas import tpu_sc as plsc

N = 10000
E = 320000
D = 128
H = 8
DH = 16

NC = 2
NS = 16
NW = NC * NS
EPT = E // NW
K = 80
NCHUNK = EPT // K
RPT = N // NS
ZROWS = 125
ACCW = D + DH


def _prep_body(f_ref, ws_ref, bs_ref, wd_ref, bd_ref, wv_ref, bv_ref,
               h_ref, fsv_ref, fd_ref):
    x = f_ref[...]
    mu = jnp.mean(x, axis=-1, keepdims=True)
    xc = x - mu
    var = jnp.mean(xc * xc, axis=-1, keepdims=True)
    h = xc * lax.rsqrt(var + 1e-5)
    dn = (((1,), (1,)), ((), ()))
    fs = lax.dot_general(h, ws_ref[...], dn,
                         preferred_element_type=jnp.float32) + bs_ref[...]
    fd = lax.dot_general(h, wd_ref[...], dn,
                         preferred_element_type=jnp.float32) + bd_ref[...]
    fv = lax.dot_general(h, wv_ref[...], dn,
                         preferred_element_type=jnp.float32) + bv_ref[...]
    h_ref[...] = h
    fsv_ref[:, 0:D] = fs
    fsv_ref[:, D:2 * D] = fv
    fd_ref[...] = fd


def _prep(feat, W_src, b_src, W_dst, b_dst, W_val, b_val):
    blk = 2000
    grid = (N // blk,)
    wspec = pl.BlockSpec((D, D), lambda i: (0, 0))
    bspec = pl.BlockSpec((1, D), lambda i: (0, 0))
    return pl.pallas_call(
        _prep_body,
        grid=grid,
        in_specs=[
            pl.BlockSpec((blk, D), lambda i: (i, 0)),
            wspec, bspec, wspec, bspec, wspec, bspec,
        ],
        out_specs=[
            pl.BlockSpec((blk, D), lambda i: (i, 0)),
            pl.BlockSpec((blk, 2 * D), lambda i: (i, 0)),
            pl.BlockSpec((blk, D), lambda i: (i, 0)),
        ],
        out_shape=[
            jax.ShapeDtypeStruct((N, D), jnp.float32),
            jax.ShapeDtypeStruct((N, 2 * D), jnp.float32),
            jax.ShapeDtypeStruct((N, D), jnp.float32),
        ],
    )(feat, W_src, b_src.reshape(1, D), W_dst, b_dst.reshape(1, D),
      W_val, b_val.reshape(1, D))


def _edge_body(fsv_hbm, fd_hbm, edge_hbm, attn_hbm, acc_hbm,
               attn_v, sidx_v, didx_v, fsv_v, fd_v, mrow_v, ebuf_v, zbuf_v,
               acc_sh, sem_a, sem_b):
    c = lax.axis_index("c")
    s = lax.axis_index("s")
    wid = c * NS + s
    zero16 = jnp.zeros((16,), jnp.float32)

    def zb(k, _):
        for j in range(ACCW // 16):
            zbuf_v[k, pl.ds(16 * j, 16)] = zero16
        return 0
    lax.fori_loop(0, ZROWS, zb, 0)

    def ze(k, _):
        ebuf_v[k, :] = zero16
        return 0
    lax.fori_loop(0, K, ze, 0)

    for j in range(RPT // ZROWS):
        pltpu.sync_copy(zbuf_v, acc_sh.at[pl.ds(s * RPT + j * ZROWS, ZROWS)])
    pltpu.sync_copy(attn_hbm, attn_v)
    plsc.subcore_barrier()

    hidx = [jnp.full((16,), h, dtype=jnp.int32) for h in range(H)]

    def chunk(ci, _):
        base = wid * EPT + ci * K
        pltpu.sync_copy(edge_hbm.at[0, pl.ds(base, K)], sidx_v)
        pltpu.sync_copy(edge_hbm.at[1, pl.ds(base, K)], didx_v)
        ga = pltpu.async_copy(fsv_hbm.at[sidx_v], fsv_v, sem_a)
        gb = pltpu.async_copy(fd_hbm.at[didx_v], fd_v, sem_b)
        ga.wait()
        gb.wait()

        def e1(k, _):
            for h in range(H):
                fs = fsv_v[k, pl.ds(h * DH, DH)]
                fdv = fd_v[k, pl.ds(h * DH, DH)]
                x = fs + fdv
                sl = x / (1.0 + jnp.exp(-x))
                t = sl * attn_v[pl.ds(h * DH, DH)]
                ebuf_v[k, h] = jnp.sum(t)
            return 0
        lax.fori_loop(0, K, e1, 0)

        def e2(k, _):
            pv = jnp.exp(ebuf_v[k, :])
            mrow_v[k, pl.ds(D, 16)] = pv
            for h in range(H):
                bv = jnp.take(pv, hidx[h], mode="promise_in_bounds")
                mrow_v[k, pl.ds(h * DH, DH)] = (
                    fsv_v[k, pl.ds(D + h * DH, DH)] * bv)
            return 0
        lax.fori_loop(0, K, e2, 0)

        pltpu.sync_copy(mrow_v, acc_sh.at[didx_v], add=True)
        return 0
    lax.fori_loop(0, NCHUNK, chunk, 0)

    plsc.subcore_barrier()
    pltpu.sync_copy(acc_sh.at[pl.ds(s * RPT, RPT)],
                    acc_hbm.at[c, pl.ds(s * RPT, RPT)])


@functools.partial(
    pl.kernel,
    out_type=jax.ShapeDtypeStruct((NC, N, ACCW), jnp.float32),
    mesh=plsc.VectorSubcoreMesh(core_axis_name="c", subcore_axis_name="s",
                                num_cores=NC, num_subcores=NS),
    scratch_types=[
        pltpu.VMEM((D,), jnp.float32),
        pltpu.VMEM((K,), jnp.int32),
        pltpu.VMEM((K,), jnp.int32),
        pltpu.VMEM((K, 2 * D), jnp.float32),
        pltpu.VMEM((K, D), jnp.float32),
        pltpu.VMEM((K, ACCW), jnp.float32),
        pltpu.VMEM((K, 16), jnp.float32),
        pltpu.VMEM((ZROWS, ACCW), jnp.float32),
        pltpu.VMEM_SHARED((N, ACCW), jnp.float32),
        pltpu.SemaphoreType.DMA,
        pltpu.SemaphoreType.DMA,
    ],
)
def _edge(fsv_hbm, fd_hbm, edge_hbm, attn_hbm, acc_hbm,
          attn_v, sidx_v, didx_v, fsv_v, fd_v, mrow_v, ebuf_v, zbuf_v,
          acc_sh, sem_a, sem_b):
    _edge_body(fsv_hbm, fd_hbm, edge_hbm, attn_hbm, acc_hbm,
               attn_v, sidx_v, didx_v, fsv_v, fd_v, mrow_v, ebuf_v, zbuf_v,
               acc_sh, sem_a, sem_b)


def _fin_body(acc_ref, h_ref, o_ref):
    a = acc_ref[...]
    numer = a[0, :, 0:D] + a[1, :, 0:D]
    d8 = a[0, :, D:D + H] + a[1, :, D:D + H]
    lane = lax.broadcasted_iota(jnp.int32, (H, D), 1)
    row = lax.broadcasted_iota(jnp.int32, (H, D), 0)
    sel = (lane // DH == row).astype(jnp.float32)
    dexp = lax.dot_general(d8, sel, (((1,), (0,)), ((), ())),
                           preferred_element_type=jnp.float32)
    dsafe = jnp.where(dexp > 0.0, dexp, 1.0)
    hu = numer / dsafe + h_ref[...]
    o_ref[...] = hu / (1.0 + jnp.exp(-hu))


def _finish(acc, h):
    blk = 2000
    grid = (N // blk,)
    return pl.pallas_call(
        _fin_body,
        grid=grid,
        in_specs=[
            pl.BlockSpec((NC, blk, ACCW), lambda i: (0, i, 0)),
            pl.BlockSpec((blk, D), lambda i: (i, 0)),
        ],
        out_specs=pl.BlockSpec((blk, D), lambda i: (i, 0)),
        out_shape=jax.ShapeDtypeStruct((N, D), jnp.float32),
    )(acc, h)


def kernel(feat, edge_index, W_src, b_src, W_dst, b_dst, W_val, b_val, attn):
    h, fsv, fd = _prep(feat, W_src, b_src, W_dst, b_dst, W_val, b_val)
    acc = _edge(fsv, fd, edge_index, attn.reshape(D))
    return _finish(acc, h)


# speedup vs baseline: 22.2283x; 22.2283x over previous
import functools

import jax
import jax.numpy as jnp
from jax import lax
from jax.experimental import pallas as pl
from jax.experimental.pallas import tpu as pltpu
from jax.experimental.pallas import tpu_sc as plsc

N = 10000
E = 320000
D = 128
H = 8
DH = 16

NC = 2
NS = 16
HC = H // NC
HW = HC * DH
EPS = E // NS
K = 80
NCHUNK = EPS // K
ACCW = HW + 16
ZCH = 200
NZCH = N // ZCH


def _prep_body(f_ref, ws_ref, bs_ref, wd_ref, bd_ref, wv_ref, bv_ref,
               h_ref, fsv_ref, fd_ref):
    x = f_ref[...]
    mu = jnp.mean(x, axis=-1, keepdims=True)
    xc = x - mu
    var = jnp.mean(xc * xc, axis=-1, keepdims=True)
    h = xc * lax.rsqrt(var + 1e-5)
    dn = (((1,), (1,)), ((), ()))
    fs = lax.dot_general(h, ws_ref[...], dn,
                         preferred_element_type=jnp.float32) + bs_ref[...]
    fd = lax.dot_general(h, wd_ref[...], dn,
                         preferred_element_type=jnp.float32) + bd_ref[...]
    fv = lax.dot_general(h, wv_ref[...], dn,
                         preferred_element_type=jnp.float32) + bv_ref[...]
    h_ref[...] = h
    fsv_ref[0] = jnp.concatenate([fs[:, 0:HW], fv[:, 0:HW]], axis=1)
    fsv_ref[1] = jnp.concatenate([fs[:, HW:D], fv[:, HW:D]], axis=1)
    fd_ref[0] = fd[:, 0:HW]
    fd_ref[1] = fd[:, HW:D]


def _prep(feat, W_src, b_src, W_dst, b_dst, W_val, b_val):
    blk = 2000
    grid = (N // blk,)
    wspec = pl.BlockSpec((D, D), lambda i: (0, 0))
    bspec = pl.BlockSpec((1, D), lambda i: (0, 0))
    return pl.pallas_call(
        _prep_body,
        grid=grid,
        in_specs=[
            pl.BlockSpec((blk, D), lambda i: (i, 0)),
            wspec, bspec, wspec, bspec, wspec, bspec,
        ],
        out_specs=[
            pl.BlockSpec((blk, D), lambda i: (i, 0)),
            pl.BlockSpec((NC, blk, 2 * HW), lambda i: (0, i, 0)),
            pl.BlockSpec((NC, blk, HW), lambda i: (0, i, 0)),
        ],
        out_shape=[
            jax.ShapeDtypeStruct((N, D), jnp.float32),
            jax.ShapeDtypeStruct((NC, N, 2 * HW), jnp.float32),
            jax.ShapeDtypeStruct((NC, N, HW), jnp.float32),
        ],
    )(feat, W_src, b_src.reshape(1, D), W_dst, b_dst.reshape(1, D),
      W_val, b_val.reshape(1, D))


def _edge_body(fsv_hbm, fd_hbm, src_hbm, dst_hbm, attn_hbm, acc_hbm,
               attn_v, sidx_v, didx_v, didx2_v, fsv_v, fd_v, mrow_v, ebuf_v, zbuf_v,
               acc_sh, sem_a, sem_b):
    c = lax.axis_index("c")
    s = lax.axis_index("s")
    zero16 = jnp.zeros((16,), jnp.float32)

    def zb(k, _):
        for j in range(ACCW // 16):
            zbuf_v[k, pl.ds(16 * j, 16)] = zero16
        return 0
    lax.fori_loop(0, ZCH, zb, 0)

    for jj in range((NZCH + NS - 1) // NS):
        idx = s + jj * NS

        @pl.when(idx < NZCH)
        def _():
            pltpu.sync_copy(zbuf_v, acc_sh.at[pl.ds(idx * ZCH, ZCH)])
    pltpu.sync_copy(attn_hbm.at[pl.ds(c * HW, HW)], attn_v)
    plsc.subcore_barrier()

    gd = lax.GatherDimensionNumbers(offset_dims=(), collapsed_slice_dims=(0,),
                                    start_index_map=(0,))
    lane = lax.iota(jnp.int32, 16)
    lane_is = [lane == h for h in range(HC)]
    perms = [(lane ^ (1 << j)).reshape(16, 1) for j in range(4)]
    hidx = [(lane * 0 + h).reshape(16, 1) for h in range(HC)]

    def lanesum(v):
        for p in perms:
            v = v + lax.gather(v, p, gd, slice_sizes=(1,),
                               mode=lax.GatherScatterMode.PROMISE_IN_BOUNDS)
        return v

    def chunk(ci, _):
        base = s * EPS + ci * K
        pltpu.sync_copy(src_hbm.at[pl.ds(base, K)], sidx_v)
        pltpu.sync_copy(dst_hbm.at[pl.ds(base, K)], didx_v)

        def shift(i, _):
            sidx_v[pl.ds(16 * i, 16)] = sidx_v[pl.ds(16 * i, 16)] + c * N
            didx2_v[pl.ds(16 * i, 16)] = didx_v[pl.ds(16 * i, 16)] + c * N
            return 0
        lax.fori_loop(0, K // 16, shift, 0)

        ga = pltpu.async_copy(fsv_hbm.at[sidx_v], fsv_v, sem_a)
        gb = pltpu.async_copy(fd_hbm.at[didx2_v], fd_v, sem_b)
        ga.wait()
        gb.wait()

        def e1(k, _):
            evec = jnp.zeros((16,), jnp.float32)
            for h in range(HC):
                fs = fsv_v[k, pl.ds(h * DH, DH)]
                fdv = fd_v[k, pl.ds(h * DH, DH)]
                x = fs + fdv
                sl = x / (1.0 + jnp.exp(-x))
                t = sl * attn_v[pl.ds(h * DH, DH)]
                evec = jnp.where(lane_is[h], lanesum(t), evec)
            ebuf_v[pl.ds(16 * k, 16)] = evec
            return 0
        lax.fori_loop(0, K, e1, 0)

        def e2(k, _):
            pv = jnp.exp(ebuf_v[pl.ds(16 * k, 16)])
            mrow_v[k, pl.ds(HW, 16)] = pv
            for h in range(HC):
                bv = lax.gather(pv, hidx[h], gd, slice_sizes=(1,),
                                mode=lax.GatherScatterMode.PROMISE_IN_BOUNDS)
                mrow_v[k, pl.ds(h * DH, DH)] = (
                    fsv_v[k, pl.ds(HW + h * DH, DH)] * bv)
            return 0
        lax.fori_loop(0, K, e2, 0)

        pltpu.sync_copy(mrow_v, acc_sh.at[didx_v], add=True)
        return 0
    lax.fori_loop(0, NCHUNK, chunk, 0)

    plsc.subcore_barrier()
    for jj in range((NZCH + NS - 1) // NS):
        idx = s + jj * NS

        @pl.when(idx < NZCH)
        def _():
            pltpu.sync_copy(acc_sh.at[pl.ds(idx * ZCH, ZCH)],
                            acc_hbm.at[c, pl.ds(idx * ZCH, ZCH)])


@functools.partial(
    pl.kernel,
    out_type=jax.ShapeDtypeStruct((NC, N, ACCW), jnp.float32),
    mesh=plsc.VectorSubcoreMesh(core_axis_name="c", subcore_axis_name="s",
                                num_cores=NC, num_subcores=NS),
    compiler_params=pltpu.CompilerParams(use_tc_tiling_on_sc=False),
    scratch_types=[
        pltpu.VMEM((HW,), jnp.float32),
        pltpu.VMEM((K,), jnp.int32),
        pltpu.VMEM((K,), jnp.int32),
        pltpu.VMEM((K,), jnp.int32),
        pltpu.VMEM((K, 2 * HW), jnp.float32),
        pltpu.VMEM((K, HW), jnp.float32),
        pltpu.VMEM((K, ACCW), jnp.float32),
        pltpu.VMEM((K * 16,), jnp.float32),
        pltpu.VMEM((ZCH, ACCW), jnp.float32),
        pltpu.VMEM_SHARED((N, ACCW), jnp.float32),
        pltpu.SemaphoreType.DMA,
        pltpu.SemaphoreType.DMA,
    ],
)
def _edge(fsv_hbm, fd_hbm, src_hbm, dst_hbm, attn_hbm, acc_hbm,
          attn_v, sidx_v, didx_v, didx2_v, fsv_v, fd_v, mrow_v, ebuf_v, zbuf_v,
          acc_sh, sem_a, sem_b):
    _edge_body(fsv_hbm, fd_hbm, src_hbm, dst_hbm, attn_hbm, acc_hbm,
               attn_v, sidx_v, didx_v, didx2_v, fsv_v, fd_v, mrow_v, ebuf_v, zbuf_v,
               acc_sh, sem_a, sem_b)


def _fin_body(acc_ref, h_ref, o_ref):
    a = acc_ref[...]
    numer = jnp.concatenate([a[0, :, 0:HW], a[1, :, 0:HW]], axis=1)
    col = lax.broadcasted_iota(jnp.int32, (HC, D), 1)
    row = lax.broadcasted_iota(jnp.int32, (HC, D), 0)
    sel_a = (col // DH == row).astype(jnp.float32)
    sel_b = (col // DH == row + HC).astype(jnp.float32)
    dexp = (
        lax.dot_general(a[0, :, HW:HW + HC], sel_a, (((1,), (0,)), ((), ())),
                        preferred_element_type=jnp.float32)
        + lax.dot_general(a[1, :, HW:HW + HC], sel_b, (((1,), (0,)), ((), ())),
                          preferred_element_type=jnp.float32))
    dsafe = jnp.where(dexp > 0.0, dexp, 1.0)
    hu = numer / dsafe + h_ref[...]
    o_ref[...] = hu / (1.0 + jnp.exp(-hu))


def _finish(acc, h):
    blk = 2000
    grid = (N // blk,)
    return pl.pallas_call(
        _fin_body,
        grid=grid,
        in_specs=[
            pl.BlockSpec((NC, blk, ACCW), lambda i: (0, i, 0)),
            pl.BlockSpec((blk, D), lambda i: (i, 0)),
        ],
        out_specs=pl.BlockSpec((blk, D), lambda i: (i, 0)),
        out_shape=jax.ShapeDtypeStruct((N, D), jnp.float32),
    )(acc, h)


def kernel(feat, edge_index, W_src, b_src, W_dst, b_dst, W_val, b_val, attn):
    h, fsv, fd = _prep(feat, W_src, b_src, W_dst, b_dst, W_val, b_val)
    acc = _edge(fsv.reshape(NC * N, 2 * HW), fd.reshape(NC * N, HW),
                edge_index[0], edge_index[1], attn.reshape(D))
    return _finish(acc, h)


# speedup vs baseline: 62.6463x; 2.8183x over previous
import functools

import jax
import jax.numpy as jnp
from jax import lax
from jax.experimental import pallas as pl
from jax.experimental.pallas import tpu as pltpu
from jax.experimental.pallas import tpu_sc as plsc

N = 10000
E = 320000
D = 128
H = 8
DH = 16

NC = 2
NS = 16
HC = H // NC
HW = HC * DH
EPS = E // NS
K = 80
NCHUNK = EPS // K
ACCW = HW + 16
ZCH = 200
NZCH = N // ZCH


def _prep_body(f_ref, ws_ref, bs_ref, wd_ref, bd_ref, wv_ref, bv_ref,
               h_ref, fsv_ref, fd_ref):
    x = f_ref[...]
    mu = jnp.mean(x, axis=-1, keepdims=True)
    xc = x - mu
    var = jnp.mean(xc * xc, axis=-1, keepdims=True)
    h = xc * lax.rsqrt(var + 1e-5)
    dn = (((1,), (1,)), ((), ()))
    fs = lax.dot_general(h, ws_ref[...], dn,
                         preferred_element_type=jnp.float32) + bs_ref[...]
    fd = lax.dot_general(h, wd_ref[...], dn,
                         preferred_element_type=jnp.float32) + bd_ref[...]
    fv = lax.dot_general(h, wv_ref[...], dn,
                         preferred_element_type=jnp.float32) + bv_ref[...]
    h_ref[...] = h
    fsv_ref[0] = jnp.concatenate([fs[:, 0:HW], fv[:, 0:HW]], axis=1)
    fsv_ref[1] = jnp.concatenate([fs[:, HW:D], fv[:, HW:D]], axis=1)
    fd_ref[0] = fd[:, 0:HW]
    fd_ref[1] = fd[:, HW:D]


def _prep(feat, W_src, b_src, W_dst, b_dst, W_val, b_val):
    blk = 2000
    grid = (N // blk,)
    wspec = pl.BlockSpec((D, D), lambda i: (0, 0))
    bspec = pl.BlockSpec((1, D), lambda i: (0, 0))
    return pl.pallas_call(
        _prep_body,
        grid=grid,
        in_specs=[
            pl.BlockSpec((blk, D), lambda i: (i, 0)),
            wspec, bspec, wspec, bspec, wspec, bspec,
        ],
        out_specs=[
            pl.BlockSpec((blk, D), lambda i: (i, 0)),
            pl.BlockSpec((NC, blk, 2 * HW), lambda i: (0, i, 0)),
            pl.BlockSpec((NC, blk, HW), lambda i: (0, i, 0)),
        ],
        out_shape=[
            jax.ShapeDtypeStruct((N, D), jnp.float32),
            jax.ShapeDtypeStruct((NC, N, 2 * HW), jnp.float32),
            jax.ShapeDtypeStruct((NC, N, HW), jnp.float32),
        ],
    )(feat, W_src, b_src.reshape(1, D), W_dst, b_dst.reshape(1, D),
      W_val, b_val.reshape(1, D))


def _edge_body(fsv_hbm, fd_hbm, src_hbm, dst_hbm, attn_hbm, acc_hbm,
               attn_v, didx_b, sidx_b, didx2_b, fsv_b, fd_b, mrow_b,
               zbuf_v, acc_sh, sem_d):
    c = lax.axis_index("c")
    s = lax.axis_index("s")
    zero16 = jnp.zeros((16,), jnp.float32)

    def zb(k, _):
        for j in range(ACCW // 16):
            zbuf_v[k, pl.ds(16 * j, 16)] = zero16
        return 0
    lax.fori_loop(0, ZCH, zb, 0)

    for jj in range((NZCH + NS - 1) // NS):
        idx = s + jj * NS

        @pl.when(idx < NZCH)
        def _():
            pltpu.sync_copy(zbuf_v, acc_sh.at[pl.ds(idx * ZCH, ZCH)])
    pltpu.sync_copy(attn_hbm.at[pl.ds(c * HW, HW)], attn_v)

    plsc.subcore_barrier()

    gd = lax.GatherDimensionNumbers(offset_dims=(), collapsed_slice_dims=(0,),
                                    start_index_map=(0,))
    lane = lax.iota(jnp.int32, 16)
    hidx = [(lane * 0 + h).reshape(16, 1) for h in range(HC)]
    pm8 = (lane ^ 8).reshape(16, 1)
    pm4 = (lane ^ 4).reshape(16, 1)
    pm2 = (lane ^ 2).reshape(16, 1)
    pm1 = (lane ^ 1).reshape(16, 1)
    psel = ((lane & 1) * 8).reshape(16, 1)
    lt8 = lane < 8
    lt2 = lane < 2

    def g(v, p):
        return lax.gather(v, p, gd, slice_sizes=(1,),
                          mode=lax.GatherScatterMode.PROMISE_IN_BOUNDS)

    def issue_gathers(ci):
        p = lax.rem(ci, 2)
        lb = ci * K
        sv, dv = sidx_b.at[p], didx2_b.at[p]
        pltpu.sync_copy(src_hbm.at[pl.ds(s * EPS + lb, K)], sv)
        pltpu.sync_copy(dst_hbm.at[pl.ds(s * EPS + lb, K)], dv)

        def fill(i, _):
            sl16 = pl.ds(16 * i, 16)
            sv[sl16] = sv[sl16] + c * N
            dv[sl16] = dv[sl16] + c * N
            return 0
        lax.fori_loop(0, K // 16, fill, 0)
        pltpu.async_copy(fsv_hbm.at[sv], fsv_b.at[p], sem_d.at[p])
        pltpu.async_copy(fd_hbm.at[dv], fd_b.at[p], sem_d.at[p])

    def wait_gathers(p):
        pltpu.make_async_copy(fsv_hbm.at[sidx_b.at[p]],
                              fsv_b.at[p], sem_d.at[p]).wait()
        pltpu.make_async_copy(fd_hbm.at[didx2_b.at[p]],
                              fd_b.at[p], sem_d.at[p]).wait()

    def wait_scatter(p):
        pltpu.make_async_copy(mrow_b.at[p], acc_sh.at[didx_b.at[p]],
                              sem_d.at[p]).wait()

    issue_gathers(0)

    def step(ci, _):
        p = lax.rem(ci, 2)

        @pl.when(ci + 1 < NCHUNK)
        def _():
            issue_gathers(ci + 1)
        wait_gathers(p)

        @pl.when(ci >= 2)
        def _():
            wait_scatter(p)
        fsv_v, fd_v, mrow_v, didx_v = (
            fsv_b.at[p], fd_b.at[p], mrow_b.at[p], didx_b.at[p])

        def unshift(i, _):
            didx_v[pl.ds(16 * i, 16)] = (
                didx2_b.at[p][pl.ds(16 * i, 16)] - c * N)
            return 0
        lax.fori_loop(0, K // 16, unshift, 0)

        a_h = [attn_v[pl.ds(h * DH, DH)] for h in range(HC)]

        def one_edge(k):
            ts = []
            for h in range(HC):
                x = fsv_v[k, pl.ds(h * DH, DH)] + fd_v[k, pl.ds(h * DH, DH)]
                sl = x / (1.0 + jnp.exp(-x))
                ts.append(sl * a_h[h])
            f = [t + g(t, pm8) for t in ts]
            mA = jnp.where(lt8, f[0], f[1])
            mB = jnp.where(lt8, f[2], f[3])
            for pm in (pm4, pm2, pm1):
                mA = mA + g(mA, pm)
                mB = mB + g(mB, pm)
            evec = jnp.where(lt2, g(mA, psel), g(mB, psel))
            pv = jnp.exp(evec)
            mrow_v[k, pl.ds(HW, 16)] = pv
            for h in range(HC):
                mrow_v[k, pl.ds(h * DH, DH)] = (
                    fsv_v[k, pl.ds(HW + h * DH, DH)] * g(pv, hidx[h]))

        @plsc.parallel_loop(0, K, step=1, unroll=8)
        def _(k):
            one_edge(k)

        pltpu.async_copy(mrow_v, acc_sh.at[didx_v], sem_d.at[p], add=True)
        return 0
    lax.fori_loop(0, NCHUNK, step, 0)
    wait_scatter(0)
    wait_scatter(1)

    plsc.subcore_barrier()
    for jj in range((NZCH + NS - 1) // NS):
        idx = s + jj * NS

        @pl.when(idx < NZCH)
        def _():
            pltpu.sync_copy(acc_sh.at[pl.ds(idx * ZCH, ZCH)],
                            acc_hbm.at[c, pl.ds(idx * ZCH, ZCH)])


@functools.partial(
    pl.kernel,
    out_type=jax.ShapeDtypeStruct((NC, N, ACCW), jnp.float32),
    mesh=plsc.VectorSubcoreMesh(core_axis_name="c", subcore_axis_name="s",
                                num_cores=NC, num_subcores=NS),
    compiler_params=pltpu.CompilerParams(use_tc_tiling_on_sc=False),
    scratch_types=[
        pltpu.VMEM((HW,), jnp.float32),
        pltpu.VMEM((2, K), jnp.int32),
        pltpu.VMEM((2, K), jnp.int32),
        pltpu.VMEM((2, K), jnp.int32),
        pltpu.VMEM((2, K, 2 * HW), jnp.float32),
        pltpu.VMEM((2, K, HW), jnp.float32),
        pltpu.VMEM((2, K, ACCW), jnp.float32),
        pltpu.VMEM((ZCH, ACCW), jnp.float32),
        pltpu.VMEM_SHARED((N, ACCW), jnp.float32),
        pltpu.SemaphoreType.DMA((2,)),
    ],
)
def _edge(fsv_hbm, fd_hbm, src_hbm, dst_hbm, attn_hbm, acc_hbm,
          attn_v, didx_b, sidx_b, didx2_b, fsv_b, fd_b, mrow_b,
          zbuf_v, acc_sh, sem_d):
    _edge_body(fsv_hbm, fd_hbm, src_hbm, dst_hbm, attn_hbm, acc_hbm,
               attn_v, didx_b, sidx_b, didx2_b, fsv_b, fd_b, mrow_b,
               zbuf_v, acc_sh, sem_d)


def _fin_body(acc_ref, h_ref, o_ref):
    a = acc_ref[...]
    numer = jnp.concatenate([a[0, :, 0:HW], a[1, :, 0:HW]], axis=1)
    col = lax.broadcasted_iota(jnp.int32, (HC, D), 1)
    row = lax.broadcasted_iota(jnp.int32, (HC, D), 0)
    sel_a = (col // DH == row).astype(jnp.float32)
    sel_b = (col // DH == row + HC).astype(jnp.float32)
    dexp = (
        lax.dot_general(a[0, :, HW:HW + HC], sel_a, (((1,), (0,)), ((), ())),
                        preferred_element_type=jnp.float32)
        + lax.dot_general(a[1, :, HW:HW + HC], sel_b, (((1,), (0,)), ((), ())),
                          preferred_element_type=jnp.float32))
    dsafe = jnp.where(dexp > 0.0, dexp, 1.0)
    hu = numer / dsafe + h_ref[...]
    o_ref[...] = hu / (1.0 + jnp.exp(-hu))


def _finish(acc, h):
    blk = 2000
    grid = (N // blk,)
    return pl.pallas_call(
        _fin_body,
        grid=grid,
        in_specs=[
            pl.BlockSpec((NC, blk, ACCW), lambda i: (0, i, 0)),
            pl.BlockSpec((blk, D), lambda i: (i, 0)),
        ],
        out_specs=pl.BlockSpec((blk, D), lambda i: (i, 0)),
        out_shape=jax.ShapeDtypeStruct((N, D), jnp.float32),
    )(acc, h)


def kernel(feat, edge_index, W_src, b_src, W_dst, b_dst, W_val, b_val, attn):
    h, fsv, fd = _prep(feat, W_src, b_src, W_dst, b_dst, W_val, b_val)
    acc = _edge(fsv.reshape(NC * N, 2 * HW), fd.reshape(NC * N, HW),
                edge_index[0], edge_index[1], attn.reshape(D))
    return _finish(acc, h)
